# baseline (speedup 1.0000x reference)
import functools

import jax
import jax.numpy as jnp
from jax import lax
from jax.experimental import pallas as pl
from jax.experimental.pallas import tpu as pltpu
from jax.experimental.pallas import tpu_sc as plsc

V = 1_000_000
D = 64
NW = 32
CT_FULL = 7812
NPAIR = V // 2

_params = pltpu.CompilerParams(
    use_tc_tiling_on_sc=True, needs_layout_passes=False)


def _mesh():
    return plsc.VectorSubcoreMesh(core_axis_name="c", subcore_axis_name="s")


def _wid():
    info = plsc.get_sparse_core_info()
    return lax.axis_index("s") * info.num_cores + lax.axis_index("c")


def _iota16():
    return lax.iota(jnp.int32, 16)


@functools.lru_cache(maxsize=None)
def _make_repack():
    @functools.partial(
        pl.kernel,
        mesh=_mesh(),
        out_type=jax.ShapeDtypeStruct((NPAIR, 128), jnp.float32),
        compiler_params=_params,
        scratch_types=[
            pltpu.VMEM((64, 128), jnp.float32),
            pltpu.VMEM((64, 128), jnp.float32),
            pltpu.VMEM((64, 128), jnp.float32),
            pltpu.VMEM((64, 128), jnp.float32),
            pltpu.VMEM((64, 64), jnp.float32),
            pltpu.VMEM((32, 128), jnp.float32),
            pltpu.SemaphoreType.DMA,
            pltpu.SemaphoreType.DMA,
            pltpu.SemaphoreType.DMA,
            pltpu.SemaphoreType.DMA,
        ],
    )
    def repack(table_hbm, scratch_hbm, src0, src1, dst0, dst1, src2, dst2,
               ssem0, ssem1, dsem0, dsem1):
        w = _wid()
        srcs, dsts = (src0, src1), (dst0, dst1)
        ssems, dsems = (ssem0, ssem1), (dsem0, dsem1)
        iota = _iota16()
        rv = tuple(iota + 16 * j for j in range(4))

        def src_copy(tc, b):
            return pltpu.make_async_copy(
                table_hbm.at[:, pl.ds(tc * 128, 128)], srcs[b], ssems[b])

        def dst_copy(tc, b):
            return pltpu.make_async_copy(
                dsts[b], scratch_hbm.at[pl.ds(tc * 64, 64)], dsems[b])

        def transpose_block(src, dst, npair2):
            def body(p2, carry):
                for dd in range(2):
                    p = 2 * p2 + dd
                    for k in range(8):
                        cvec = rv[0] * 0 + (2 * p + (1 if k >= 4 else 0))
                        g = plsc.load_gather(src, [rv[k % 4], cvec])
                        dst[p, pl.ds(16 * k, 16)] = g
                return carry

            lax.fori_loop(0, npair2, body, 0)

        src_copy(w, 0).start()
        src_copy(32 + w, 1).start()

        def group(q0, carry):
            for b in range(2):
                q = 2 * q0 + b
                tc = 32 * q + w

                @pl.when(tc < CT_FULL)
                def _():
                    src_copy(tc, b).wait()

                    @pl.when(q >= 2)
                    def _():
                        dst_copy(tc - 64, b).wait()

                    transpose_block(srcs[b], dsts[b], 32)
                    dst_copy(tc, b).start()

                    @pl.when(tc + 64 < CT_FULL)
                    def _():
                        src_copy(tc + 64, b).start()

            return carry

        lax.fori_loop(0, 123, group, 0)

        qmax = lax.shift_right_logical(CT_FULL - 1 - w, 5)
        for b in range(2):
            qb = jnp.where(qmax % 2 == b, qmax, qmax - 1)

            @pl.when(qb >= 0)
            def _():
                dst_copy(32 * qb + w, b).wait()

        @pl.when(w == 4)
        def _():
            pltpu.sync_copy(table_hbm.at[:, pl.ds(128 * CT_FULL, 64)], src2)

            def body(p2, carry):
                for dd in range(2):
                    p = 2 * p2 + dd
                    for k in range(8):
                        cvec = rv[0] * 0 + (2 * p + (1 if k >= 4 else 0))
                        g = plsc.load_gather(src2, [rv[k % 4], cvec])
                        dst2[p, pl.ds(16 * k, 16)] = g
                return carry

            lax.fori_loop(0, 16, body, 0)
            pltpu.sync_copy(dst2, scratch_hbm.at[pl.ds(64 * CT_FULL, 32)])

    return repack


@functools.lru_cache(maxsize=None)
def _make_gather(L, B):
    n_sb = (L // 8) * (B // 128)
    n_q = (n_sb + NW - 1) // NW

    @functools.partial(
        pl.kernel,
        mesh=_mesh(),
        out_type=jax.ShapeDtypeStruct((L, D, B), jnp.float32),
        compiler_params=_params,
        scratch_types=[
            pltpu.VMEM((8, 128), jnp.int32),
            pltpu.VMEM((128,), jnp.int32),
            pltpu.VMEM((128,), jnp.int32),
            pltpu.VMEM((128, 128), jnp.float32),
            pltpu.VMEM((128, 128), jnp.float32),
            pltpu.VMEM((64, 128), jnp.float32),
            pltpu.VMEM((64, 128), jnp.float32),
            pltpu.SemaphoreType.DMA,
            pltpu.SemaphoreType.DMA,
            pltpu.SemaphoreType.DMA,
            pltpu.SemaphoreType.DMA,
        ],
    )
    def gather(scratch_hbm, ids_hbm, out_hbm, ids_v, idx0, idx1,
               g0, g1, o0, o1, gsem0, gsem1, osem0, osem1):
        w = _wid()
        idxs, gbufs, obufs = (idx0, idx1), (g0, g1), (o0, o1)
        gsems, osems = (gsem0, gsem1), (osem0, osem1)
        iota = _iota16()
        t_base = tuple(iota + 16 * m for m in range(8))

        def prep_idx(i, bi):
            for m in range(8):
                v = ids_v[i, pl.ds(16 * m, 16)]
                idxs[bi][pl.ds(16 * m, 16)] = lax.shift_right_logical(v, 1)

        def gather_copy(bi):
            return pltpu.make_async_copy(
                scratch_hbm.at[idxs[bi]], gbufs[bi], gsems[bi])

        def out_copy(l, bb, bi):
            return pltpu.make_async_copy(
                obufs[bi], out_hbm.at[l, :, pl.ds(bb * 128, 128)], osems[bi])

        def compute(i, bi):
            pc = tuple(
                lax.shift_left(ids_v[i, pl.ds(16 * m, 16)] & 1, 6)
                for m in range(8))

            def body(d2, carry):
                for dd in range(2):
                    d = 2 * d2 + dd
                    for m in range(8):
                        g = plsc.load_gather(
                            gbufs[bi], [t_base[m], pc[m] + d])
                        obufs[bi][d, pl.ds(16 * m, 16)] = g
                return carry

            lax.fori_loop(0, 32, body, 0)

        def sb_body(q, carry):
            sb = 32 * q + w

            @pl.when(sb < n_sb)
            def _():
                l8 = lax.shift_right_logical(sb, 3)
                bb = sb & 7
                psb = sb - 32
                pl8 = lax.shift_right_logical(psb, 3)
                pbb = psb & 7
                pltpu.sync_copy(
                    ids_hbm.at[pl.ds(l8 * 8, 8), pl.ds(bb * 128, 128)], ids_v)
                prep_idx(0, 0)
                gather_copy(0).start()
                for i in range(8):
                    bi = i % 2
                    gather_copy(bi).wait()
                    if i < 7:
                        prep_idx(i + 1, 1 - bi)
                        gather_copy(1 - bi).start()
                    if i >= 2:
                        out_copy(l8 * 8 + i - 2, bb, bi).wait()
                    else:

                        @pl.when(q > 0)
                        def _():
                            out_copy(pl8 * 8 + 6 + i, pbb, bi).wait()

                    compute(i, bi)
                    out_copy(l8 * 8 + i, bb, bi).start()

            return carry

        lax.fori_loop(0, n_q, sb_body, 0)

        qlast = lax.shift_right_logical(n_sb - 1 - w, 5)
        lsb = 32 * qlast + w
        ll8 = lax.shift_right_logical(lsb, 3)
        lbb = lsb & 7
        for b in range(2):
            out_copy(ll8 * 8 + 6 + b, lbb, b).wait()

    return gather


def kernel(ids, emb_weight):
    batch, length = ids.shape
    table_t = emb_weight.T
    ids_t = ids.T
    scratch = _make_repack()(table_t)
    out_t = _make_gather(length, batch)(scratch, ids_t)
    return jnp.transpose(out_t, (2, 0, 1))


# speedup vs baseline: 1.8540x; 1.8540x over previous
import functools

import jax
import jax.numpy as jnp
from jax import lax
from jax.experimental import pallas as pl
from jax.experimental.pallas import tpu as pltpu
from jax.experimental.pallas import tpu_sc as plsc

V = 1_000_000
D = 64
NW = 32
CT_FULL = 7812
NPAIR = V // 2
SPAD = 128

_params = pltpu.CompilerParams(
    use_tc_tiling_on_sc=True, needs_layout_passes=False)


def _mesh():
    return plsc.VectorSubcoreMesh(core_axis_name="c", subcore_axis_name="s")


def _wid():
    info = plsc.get_sparse_core_info()
    return lax.axis_index("s") * info.num_cores + lax.axis_index("c")


def _iota16():
    return lax.iota(jnp.int32, 16)


@functools.lru_cache(maxsize=None)
def _make_repack():
    @functools.partial(
        pl.kernel,
        mesh=_mesh(),
        out_type=jax.ShapeDtypeStruct((NPAIR, 128), jnp.float32),
        compiler_params=_params,
        scratch_types=[
            pltpu.VMEM((64, SPAD), jnp.float32),
            pltpu.VMEM((64, SPAD), jnp.float32),
            pltpu.VMEM((64, 128), jnp.float32),
            pltpu.VMEM((64, 128), jnp.float32),
            pltpu.VMEM((64, 64), jnp.float32),
            pltpu.VMEM((32, 128), jnp.float32),
            pltpu.SemaphoreType.DMA,
            pltpu.SemaphoreType.DMA,
            pltpu.SemaphoreType.DMA,
            pltpu.SemaphoreType.DMA,
        ],
    )
    def repack(table_hbm, scratch_hbm, src0, src1, dst0, dst1, src2, dst2,
               ssem0, ssem1, dsem0, dsem1):
        w = _wid()
        srcs, dsts = (src0, src1), (dst0, dst1)
        ssems, dsems = (ssem0, ssem1), (dsem0, dsem1)
        iota = _iota16()
        rv = tuple(iota + 16 * j for j in range(4))

        def src_copy(tc, b):
            return pltpu.make_async_copy(
                table_hbm.at[:, pl.ds(tc * 128, 128)],
                srcs[b], ssems[b])

        def dst_copy(tc, b):
            return pltpu.make_async_copy(
                dsts[b], scratch_hbm.at[pl.ds(tc * 64, 64)], dsems[b])

        def transpose_block(src, dst, npair):
            @plsc.parallel_loop(0, npair, unroll=4)
            def _(p):
                for k in range(8):
                    cvec = rv[0] * 0 + (2 * p + (1 if k >= 4 else 0))
                    g = plsc.load_gather(src, [rv[k % 4], cvec])
                    dst[p, pl.ds(16 * k, 16)] = g

        src_copy(w, 0).start()
        src_copy(32 + w, 1).start()

        def group(q0, carry):
            for b in range(2):
                q = 2 * q0 + b
                tc = 32 * q + w

                @pl.when(tc < CT_FULL)
                def _():
                    src_copy(tc, b).wait()

                    @pl.when(q >= 2)
                    def _():
                        dst_copy(tc - 64, b).wait()

                    transpose_block(srcs[b], dsts[b], 64)
                    dst_copy(tc, b).start()

                    @pl.when(tc + 64 < CT_FULL)
                    def _():
                        src_copy(tc + 64, b).start()

            return carry

        lax.fori_loop(0, 123, group, 0)

        qmax = lax.shift_right_logical(CT_FULL - 1 - w, 5)
        for b in range(2):
            qb = jnp.where(qmax % 2 == b, qmax, qmax - 1)

            @pl.when(qb >= 0)
            def _():
                dst_copy(32 * qb + w, b).wait()

        @pl.when(w == 4)
        def _():
            pltpu.sync_copy(
                table_hbm.at[:, pl.ds(128 * CT_FULL, 64)],
                src2)

            @plsc.parallel_loop(0, 32, unroll=4)
            def _(p):
                for k in range(8):
                    cvec = rv[0] * 0 + (2 * p + (1 if k >= 4 else 0))
                    g = plsc.load_gather(src2, [rv[k % 4], cvec])
                    dst2[p, pl.ds(16 * k, 16)] = g

            pltpu.sync_copy(dst2, scratch_hbm.at[pl.ds(64 * CT_FULL, 32)])

    return repack


@functools.lru_cache(maxsize=None)
def _make_gather(L, B):
    n_sb = (L // 8) * (B // 128)
    n_q = (n_sb + NW - 1) // NW

    @functools.partial(
        pl.kernel,
        mesh=_mesh(),
        out_type=jax.ShapeDtypeStruct((L, D, B), jnp.float32),
        compiler_params=_params,
        scratch_types=[
            pltpu.VMEM((8, 128), jnp.int32),
            pltpu.VMEM((128,), jnp.int32),
            pltpu.VMEM((128,), jnp.int32),
            pltpu.VMEM((128, SPAD), jnp.float32),
            pltpu.VMEM((128, SPAD), jnp.float32),
            pltpu.VMEM((64, 128), jnp.float32),
            pltpu.VMEM((64, 128), jnp.float32),
            pltpu.SemaphoreType.DMA,
            pltpu.SemaphoreType.DMA,
            pltpu.SemaphoreType.DMA,
            pltpu.SemaphoreType.DMA,
        ],
    )
    def gather(scratch_hbm, ids_hbm, out_hbm, ids_v, idx0, idx1,
               g0, g1, o0, o1, gsem0, gsem1, osem0, osem1):
        w = _wid()
        idxs, gbufs, obufs = (idx0, idx1), (g0, g1), (o0, o1)
        gsems, osems = (gsem0, gsem1), (osem0, osem1)
        iota = _iota16()
        t_base = tuple(iota + 16 * m for m in range(8))

        def prep_idx(i, bi):
            for m in range(8):
                v = ids_v[i, pl.ds(16 * m, 16)]
                idxs[bi][pl.ds(16 * m, 16)] = lax.shift_right_logical(v, 1)

        def gather_copy(bi):
            return pltpu.make_async_copy(
                scratch_hbm.at[idxs[bi]],
                gbufs[bi], gsems[bi])

        def out_copy(l, bb, bi):
            return pltpu.make_async_copy(
                obufs[bi], out_hbm.at[l, :, pl.ds(bb * 128, 128)], osems[bi])

        def compute(i, bi):
            pc = tuple(
                lax.shift_left(ids_v[i, pl.ds(16 * m, 16)] & 1, 6)
                for m in range(8))

            @plsc.parallel_loop(0, 64, unroll=4)
            def _(d):
                for m in range(8):
                    g = plsc.load_gather(
                        gbufs[bi], [t_base[m], pc[m] + d])
                    obufs[bi][d, pl.ds(16 * m, 16)] = g

        def sb_body(q, carry):
            sb = 32 * q + w

            @pl.when(sb < n_sb)
            def _():
                l8 = lax.shift_right_logical(sb, 3)
                bb = sb & 7
                psb = sb - 32
                pl8 = lax.shift_right_logical(psb, 3)
                pbb = psb & 7
                pltpu.sync_copy(
                    ids_hbm.at[pl.ds(l8 * 8, 8), pl.ds(bb * 128, 128)], ids_v)
                prep_idx(0, 0)
                gather_copy(0).start()
                for i in range(8):
                    bi = i % 2
                    gather_copy(bi).wait()
                    if i < 7:
                        prep_idx(i + 1, 1 - bi)
                        gather_copy(1 - bi).start()
                    if i >= 2:
                        out_copy(l8 * 8 + i - 2, bb, bi).wait()
                    else:

                        @pl.when(q > 0)
                        def _():
                            out_copy(pl8 * 8 + 6 + i, pbb, bi).wait()

                    compute(i, bi)
                    out_copy(l8 * 8 + i, bb, bi).start()

            return carry

        lax.fori_loop(0, n_q, sb_body, 0)

        qlast = lax.shift_right_logical(n_sb - 1 - w, 5)
        lsb = 32 * qlast + w
        ll8 = lax.shift_right_logical(lsb, 3)
        lbb = lsb & 7
        for b in range(2):
            out_copy(ll8 * 8 + 6 + b, lbb, b).wait()

    return gather


def kernel(ids, emb_weight):
    batch, length = ids.shape
    table_t = emb_weight.T
    ids_t = ids.T
    scratch = _make_repack()(table_t)
    out_t = _make_gather(length, batch)(scratch, ids_t)
    return jnp.transpose(out_t, (2, 0, 1))


# speedup vs baseline: 5.5410x; 2.9888x over previous
import functools

import jax
import jax.numpy as jnp
from jax import lax
from jax.experimental import pallas as pl
from jax.experimental.pallas import tpu as pltpu
from jax.experimental.pallas import tpu_sc as plsc

V = 1_000_000
D = 64
NW = 32
CT_FULL = 7812
NPAIR = V // 2
SPAD = 128

_params = pltpu.CompilerParams(
    use_tc_tiling_on_sc=True, needs_layout_passes=False)


def _mesh():
    return plsc.VectorSubcoreMesh(core_axis_name="c", subcore_axis_name="s")


def _wid():
    info = plsc.get_sparse_core_info()
    return lax.axis_index("s") * info.num_cores + lax.axis_index("c")


def _iota16():
    return lax.iota(jnp.int32, 16)


@functools.lru_cache(maxsize=None)
def _make_repack():
    @functools.partial(
        pl.kernel,
        mesh=_mesh(),
        out_type=jax.ShapeDtypeStruct((NPAIR, 128), jnp.float32),
        compiler_params=_params,
        scratch_types=[
            pltpu.VMEM((64, SPAD), jnp.float32),
            pltpu.VMEM((64, SPAD), jnp.float32),
            pltpu.VMEM((64, 128), jnp.float32),
            pltpu.VMEM((64, 128), jnp.float32),
            pltpu.VMEM((64, 64), jnp.float32),
            pltpu.VMEM((32, 128), jnp.float32),
            pltpu.SemaphoreType.DMA,
            pltpu.SemaphoreType.DMA,
            pltpu.SemaphoreType.DMA,
            pltpu.SemaphoreType.DMA,
        ],
    )
    def repack(table_hbm, scratch_hbm, src0, src1, dst0, dst1, src2, dst2,
               ssem0, ssem1, dsem0, dsem1):
        w = _wid()
        srcs, dsts = (src0, src1), (dst0, dst1)
        ssems, dsems = (ssem0, ssem1), (dsem0, dsem1)
        iota = _iota16()

        def src_copy(tc, b):
            return pltpu.make_async_copy(
                table_hbm.at[:, pl.ds(tc * 128, 128)],
                srcs[b], ssems[b])

        def dst_copy(tc, b):
            return pltpu.make_async_copy(
                dsts[b], scratch_hbm.at[pl.ds(tc * 64, 64)], dsems[b])

        def transpose_block(src, dst, npair):
            pvs = tuple(iota + 16 * pb for pb in range(npair // 16))

            @plsc.parallel_loop(0, 128, unroll=2)
            def _(c0):
                c_vec = (c0 + iota) & 127
                r_vec = c_vec & 63
                par = lax.shift_right_logical(c_vec, 6)
                for pb in range(npair // 16):
                    t_vec = lax.shift_left(pvs[pb], 1) + par
                    g = plsc.load_gather(src, [r_vec, t_vec])
                    plsc.store_scatter(dst, [pvs[pb], c_vec], g)

        src_copy(w, 0).start()
        src_copy(32 + w, 1).start()

        def group(q0, carry):
            for b in range(2):
                q = 2 * q0 + b
                tc = 32 * q + w

                @pl.when(tc < CT_FULL)
                def _():
                    src_copy(tc, b).wait()

                    @pl.when(q >= 2)
                    def _():
                        dst_copy(tc - 64, b).wait()

                    transpose_block(srcs[b], dsts[b], 64)
                    dst_copy(tc, b).start()

                    @pl.when(tc + 64 < CT_FULL)
                    def _():
                        src_copy(tc + 64, b).start()

            return carry

        lax.fori_loop(0, 123, group, 0)

        qmax = lax.shift_right_logical(CT_FULL - 1 - w, 5)
        for b in range(2):
            qb = jnp.where(qmax % 2 == b, qmax, qmax - 1)

            @pl.when(qb >= 0)
            def _():
                dst_copy(32 * qb + w, b).wait()

        @pl.when(w == 4)
        def _():
            pltpu.sync_copy(
                table_hbm.at[:, pl.ds(128 * CT_FULL, 64)],
                src2)

            pvs2 = (iota, iota + 16)

            @plsc.parallel_loop(0, 128, unroll=2)
            def _(c0):
                c_vec = (c0 + iota) & 127
                r_vec = c_vec & 63
                par = lax.shift_right_logical(c_vec, 6)
                for pb in range(2):
                    t_vec = lax.shift_left(pvs2[pb], 1) + par
                    g = plsc.load_gather(src2, [r_vec, t_vec])
                    plsc.store_scatter(dst2, [pvs2[pb], c_vec], g)

            pltpu.sync_copy(dst2, scratch_hbm.at[pl.ds(64 * CT_FULL, 32)])

    return repack


@functools.lru_cache(maxsize=None)
def _make_gather(L, B):
    n_sb = (L // 8) * (B // 128)
    n_q = (n_sb + NW - 1) // NW

    @functools.partial(
        pl.kernel,
        mesh=_mesh(),
        out_type=jax.ShapeDtypeStruct((L, D, B), jnp.float32),
        compiler_params=_params,
        scratch_types=[
            pltpu.VMEM((8, 128), jnp.int32),
            pltpu.VMEM((128,), jnp.int32),
            pltpu.VMEM((128,), jnp.int32),
            pltpu.VMEM((128,), jnp.int32),
            pltpu.VMEM((128,), jnp.int32),
            pltpu.VMEM((128, SPAD), jnp.float32),
            pltpu.VMEM((128, SPAD), jnp.float32),
            pltpu.VMEM((64, 128), jnp.float32),
            pltpu.VMEM((64, 128), jnp.float32),
            pltpu.SemaphoreType.DMA,
            pltpu.SemaphoreType.DMA,
            pltpu.SemaphoreType.DMA,
            pltpu.SemaphoreType.DMA,
        ],
    )
    def gather(scratch_hbm, ids_hbm, out_hbm, ids_v, idx0, idx1, par0, par1,
               g0, g1, o0, o1, gsem0, gsem1, osem0, osem1):
        w = _wid()
        idxs, pars = (idx0, idx1), (par0, par1)
        gbufs, obufs = (g0, g1), (o0, o1)
        gsems, osems = (gsem0, gsem1), (osem0, osem1)
        iota = _iota16()
        t_base = tuple(iota + 16 * m for m in range(8))

        def prep_idx(i, bi):
            for m in range(8):
                v = ids_v[i, pl.ds(16 * m, 16)]
                idxs[bi][pl.ds(16 * m, 16)] = lax.shift_right_logical(v, 1)
                pars[bi][pl.ds(16 * m, 16)] = lax.shift_left(v & 1, 6)

        def gather_copy(bi):
            return pltpu.make_async_copy(
                scratch_hbm.at[idxs[bi]],
                gbufs[bi], gsems[bi])

        def out_copy(l, bb, bi):
            return pltpu.make_async_copy(
                obufs[bi], out_hbm.at[l, :, pl.ds(bb * 128, 128)], osems[bi])

        def compute(i, bi):
            pc = tuple(pars[bi][pl.ds(16 * m, 16)] for m in range(8))

            @plsc.parallel_loop(0, 64, unroll=2)
            def _(d0):
                d_vec = (d0 + iota) & 63
                for m in range(8):
                    g = plsc.load_gather(
                        gbufs[bi], [t_base[m], pc[m] + d_vec])
                    plsc.store_scatter(obufs[bi], [d_vec, t_base[m]], g)

        def sb_body(q, carry):
            sb = 32 * q + w

            @pl.when(sb < n_sb)
            def _():
                l8 = lax.shift_right_logical(sb, 3)
                bb = sb & 7
                psb = sb - 32
                pl8 = lax.shift_right_logical(psb, 3)
                pbb = psb & 7
                pltpu.sync_copy(
                    ids_hbm.at[pl.ds(l8 * 8, 8), pl.ds(bb * 128, 128)], ids_v)
                prep_idx(0, 0)
                gather_copy(0).start()
                for i in range(8):
                    bi = i % 2
                    gather_copy(bi).wait()
                    if i < 7:
                        prep_idx(i + 1, 1 - bi)
                        gather_copy(1 - bi).start()
                    if i >= 2:
                        out_copy(l8 * 8 + i - 2, bb, bi).wait()
                    else:

                        @pl.when(q > 0)
                        def _():
                            out_copy(pl8 * 8 + 6 + i, pbb, bi).wait()

                    compute(i, bi)
                    out_copy(l8 * 8 + i, bb, bi).start()

            return carry

        lax.fori_loop(0, n_q, sb_body, 0)

        qlast = lax.shift_right_logical(n_sb - 1 - w, 5)
        lsb = 32 * qlast + w
        ll8 = lax.shift_right_logical(lsb, 3)
        lbb = lsb & 7
        for b in range(2):
            out_copy(ll8 * 8 + 6 + b, lbb, b).wait()

    return gather


def kernel(ids, emb_weight):
    batch, length = ids.shape
    table_t = emb_weight.T
    ids_t = ids.T
    scratch = _make_repack()(table_t)
    out_t = _make_gather(length, batch)(scratch, ids_t)
    return jnp.transpose(out_t, (2, 0, 1))


# speedup vs baseline: 5.9039x; 1.0655x over previous
import functools

import jax
import jax.numpy as jnp
from jax import lax
from jax.experimental import pallas as pl
from jax.experimental.pallas import tpu as pltpu
from jax.experimental.pallas import tpu_sc as plsc

V = 1_000_000
D = 64
NW = 32
CT_FULL = 7812
NPAIR = V // 2
SPAD = 128

_params = pltpu.CompilerParams(
    use_tc_tiling_on_sc=True, needs_layout_passes=False)


def _mesh():
    return plsc.VectorSubcoreMesh(core_axis_name="c", subcore_axis_name="s")


def _wid():
    info = plsc.get_sparse_core_info()
    return lax.axis_index("s") * info.num_cores + lax.axis_index("c")


def _iota16():
    return lax.iota(jnp.int32, 16)


@functools.lru_cache(maxsize=None)
def _make_repack():
    @functools.partial(
        pl.kernel,
        mesh=_mesh(),
        out_type=jax.ShapeDtypeStruct((NPAIR, 128), jnp.float32),
        compiler_params=_params,
        scratch_types=[
            pltpu.VMEM((64, SPAD), jnp.float32),
            pltpu.VMEM((64, SPAD), jnp.float32),
            pltpu.VMEM((64, 128), jnp.float32),
            pltpu.VMEM((64, 128), jnp.float32),
            pltpu.VMEM((64, 64), jnp.float32),
            pltpu.VMEM((32, 128), jnp.float32),
            pltpu.SemaphoreType.DMA,
            pltpu.SemaphoreType.DMA,
            pltpu.SemaphoreType.DMA,
            pltpu.SemaphoreType.DMA,
        ],
    )
    def repack(table_hbm, scratch_hbm, src0, src1, dst0, dst1, src2, dst2,
               ssem0, ssem1, dsem0, dsem1):
        w = _wid()
        srcs, dsts = (src0, src1), (dst0, dst1)
        ssems, dsems = (ssem0, ssem1), (dsem0, dsem1)
        iota = _iota16()

        def src_copy(tc, b):
            return pltpu.make_async_copy(
                table_hbm.at[:, pl.ds(tc * 128, 128)],
                srcs[b], ssems[b])

        def dst_copy(tc, b):
            return pltpu.make_async_copy(
                dsts[b], scratch_hbm.at[pl.ds(tc * 64, 64)], dsems[b])

        def transpose_block(src, dst, npair):
            pvs = tuple(iota + 16 * pb for pb in range(npair // 16))
            tb2 = tuple(lax.shift_left(pv, 1) for pv in pvs)

            @plsc.parallel_loop(0, 128, unroll=4)
            def _(c0):
                c_vec = (c0 + iota) & 127
                r_vec = c_vec & 63
                par = lax.shift_right_logical(c_vec, 6)
                for pb in range(npair // 16):
                    g = plsc.load_gather(src, [r_vec, tb2[pb] + par])
                    plsc.store_scatter(dst, [pvs[pb], c_vec], g)

        src_copy(w, 0).start()
        src_copy(32 + w, 1).start()

        def group(q0, carry):
            for b in range(2):
                q = 2 * q0 + b
                tc = 32 * q + w

                @pl.when(tc < CT_FULL)
                def _():
                    src_copy(tc, b).wait()

                    @pl.when(q >= 2)
                    def _():
                        dst_copy(tc - 64, b).wait()

                    transpose_block(srcs[b], dsts[b], 64)
                    dst_copy(tc, b).start()

                    @pl.when(tc + 64 < CT_FULL)
                    def _():
                        src_copy(tc + 64, b).start()

            return carry

        lax.fori_loop(0, 123, group, 0)

        qmax = lax.shift_right_logical(CT_FULL - 1 - w, 5)
        for b in range(2):
            qb = jnp.where(qmax % 2 == b, qmax, qmax - 1)

            @pl.when(qb >= 0)
            def _():
                dst_copy(32 * qb + w, b).wait()

        @pl.when(w == 4)
        def _():
            pltpu.sync_copy(
                table_hbm.at[:, pl.ds(128 * CT_FULL, 64)],
                src2)

            pvs2 = (iota, iota + 16)

            @plsc.parallel_loop(0, 128, unroll=2)
            def _(c0):
                c_vec = (c0 + iota) & 127
                r_vec = c_vec & 63
                par = lax.shift_right_logical(c_vec, 6)
                for pb in range(2):
                    t_vec = lax.shift_left(pvs2[pb], 1) + par
                    g = plsc.load_gather(src2, [r_vec, t_vec])
                    plsc.store_scatter(dst2, [pvs2[pb], c_vec], g)

            pltpu.sync_copy(dst2, scratch_hbm.at[pl.ds(64 * CT_FULL, 32)])

    return repack


@functools.lru_cache(maxsize=None)
def _make_gather(L, B):
    n_sb = (L // 8) * (B // 128)
    n_q = (n_sb + NW - 1) // NW

    @functools.partial(
        pl.kernel,
        mesh=_mesh(),
        out_type=jax.ShapeDtypeStruct((L, D, B), jnp.float32),
        compiler_params=_params,
        scratch_types=[
            pltpu.VMEM((8, 128), jnp.int32),
            pltpu.VMEM((8, 128), jnp.int32),
            pltpu.VMEM((8, 128), jnp.int32),
            pltpu.VMEM((128, 128), jnp.float32),
            pltpu.VMEM((128, 128), jnp.float32),
            pltpu.VMEM((128, 128), jnp.float32),
            pltpu.VMEM((64, 128), jnp.float32),
            pltpu.VMEM((64, 128), jnp.float32),
            pltpu.SemaphoreType.DMA,
            pltpu.SemaphoreType.DMA,
            pltpu.SemaphoreType.DMA,
            pltpu.SemaphoreType.DMA,
            pltpu.SemaphoreType.DMA,
        ],
    )
    def gather(scratch_hbm, ids_hbm, out_hbm, ids_v, idx2, par2,
               g0, g1, g2, o0, o1, gsem0, gsem1, gsem2, osem0, osem1):
        w = _wid()
        gbufs, obufs = (g0, g1, g2), (o0, o1)
        gsems, osems = (gsem0, gsem1, gsem2), (osem0, osem1)
        iota = _iota16()
        t_base = tuple(iota + 16 * m for m in range(8))

        def prep_all():
            @plsc.parallel_loop(0, 8)
            def _(i):
                for m in range(8):
                    v = ids_v[i, pl.ds(16 * m, 16)]
                    idx2[i, pl.ds(16 * m, 16)] = lax.shift_right_logical(v, 1)
                    par2[i, pl.ds(16 * m, 16)] = lax.shift_left(v & 1, 6)

        def gather_copy(i, bi):
            return pltpu.make_async_copy(
                scratch_hbm.at[idx2.at[i]], gbufs[bi], gsems[bi])

        def out_copy(l, bb, bi):
            return pltpu.make_async_copy(
                obufs[bi], out_hbm.at[l, :, pl.ds(bb * 128, 128)], osems[bi])

        def compute(i, bi, oi):
            pc = tuple(par2[i, pl.ds(16 * m, 16)] for m in range(8))

            @plsc.parallel_loop(0, 64, unroll=4)
            def _(d0):
                d_vec = (d0 + iota) & 63
                for m in range(8):
                    g = plsc.load_gather(
                        gbufs[bi], [t_base[m], pc[m] + d_vec])
                    plsc.store_scatter(obufs[oi], [d_vec, t_base[m]], g)

        def sb_body(q, carry):
            sb = 32 * q + w

            @pl.when(sb < n_sb)
            def _():
                l8 = lax.shift_right_logical(sb, 3)
                bb = sb & 7
                psb = sb - 32
                pl8 = lax.shift_right_logical(psb, 3)
                pbb = psb & 7
                pltpu.sync_copy(
                    ids_hbm.at[pl.ds(l8 * 8, 8), pl.ds(bb * 128, 128)], ids_v)
                prep_all()
                for i in range(3):
                    gather_copy(i, i).start()
                for i in range(8):
                    bi = i % 3
                    oi = i % 2
                    gather_copy(i, bi).wait()
                    if i >= 2:
                        out_copy(l8 * 8 + i - 2, bb, oi).wait()
                    else:

                        @pl.when(q > 0)
                        def _():
                            out_copy(pl8 * 8 + 6 + i, pbb, oi).wait()

                    compute(i, bi, oi)
                    out_copy(l8 * 8 + i, bb, oi).start()
                    if i + 3 < 8:
                        gather_copy(i + 3, bi).start()

            return carry

        lax.fori_loop(0, n_q, sb_body, 0)

        qlast = lax.shift_right_logical(n_sb - 1 - w, 5)
        lsb = 32 * qlast + w
        ll8 = lax.shift_right_logical(lsb, 3)
        lbb = lsb & 7
        for b in range(2):
            out_copy(ll8 * 8 + 6 + b, lbb, b).wait()

    return gather


def kernel(ids, emb_weight):
    batch, length = ids.shape
    table_t = emb_weight.T
    ids_t = ids.T
    scratch = _make_repack()(table_t)
    out_t = _make_gather(length, batch)(scratch, ids_t)
    return jnp.transpose(out_t, (2, 0, 1))


# speedup vs baseline: 6.6910x; 1.1333x over previous
import functools

import jax
import jax.numpy as jnp
from jax import lax
from jax.experimental import pallas as pl
from jax.experimental.pallas import tpu as pltpu
from jax.experimental.pallas import tpu_sc as plsc

V = 1_000_000
D = 64
NW = 32
CT_FULL = 7812
NPAIR = V // 2
SPAD = 128

_params = pltpu.CompilerParams(
    use_tc_tiling_on_sc=True, needs_layout_passes=False)


def _mesh():
    return plsc.VectorSubcoreMesh(core_axis_name="c", subcore_axis_name="s")


def _wid():
    info = plsc.get_sparse_core_info()
    return lax.axis_index("s") * info.num_cores + lax.axis_index("c")


def _iota16():
    return lax.iota(jnp.int32, 16)


@functools.lru_cache(maxsize=None)
def _make_repack():
    NDC = CT_FULL // 2

    @functools.partial(
        pl.kernel,
        mesh=_mesh(),
        out_type=jax.ShapeDtypeStruct((NPAIR, 128), jnp.float32),
        compiler_params=_params,
        scratch_types=[
            pltpu.VMEM((64, 256), jnp.float32),
            pltpu.VMEM((64, 256), jnp.float32),
            pltpu.VMEM((64, 256), jnp.float32),
            pltpu.VMEM((128, 128), jnp.float32),
            pltpu.VMEM((128, 128), jnp.float32),
            pltpu.VMEM((128, 128), jnp.float32),
            pltpu.VMEM((64, 64), jnp.float32),
            pltpu.VMEM((32, 128), jnp.float32),
            pltpu.SemaphoreType.DMA,
            pltpu.SemaphoreType.DMA,
            pltpu.SemaphoreType.DMA,
            pltpu.SemaphoreType.DMA,
            pltpu.SemaphoreType.DMA,
            pltpu.SemaphoreType.DMA,
        ],
    )
    def repack(table_hbm, scratch_hbm, src0, src1, src2a, dst0, dst1, dst2a,
               srcp, dstp, ssem0, ssem1, ssem2, dsem0, dsem1, dsem2):
        w = _wid()
        srcs, dsts = (src0, src1, src2a), (dst0, dst1, dst2a)
        ssems, dsems = (ssem0, ssem1, ssem2), (dsem0, dsem1, dsem2)
        iota = _iota16()

        def src_copy(dc, b):
            return pltpu.make_async_copy(
                table_hbm.at[:, pl.ds(dc * 256, 256)], srcs[b], ssems[b])

        def dst_copy(dc, b):
            return pltpu.make_async_copy(
                dsts[b], scratch_hbm.at[pl.ds(dc * 128, 128)], dsems[b])

        pvs = tuple(iota + 16 * pb for pb in range(8))
        tb2 = tuple(lax.shift_left(pv, 1) for pv in pvs)

        def transpose_block(src, dst):
            @plsc.parallel_loop(0, 128, unroll=4)
            def _(c0):
                c_vec = (c0 + iota) & 127
                r_vec = c_vec & 63
                par = lax.shift_right_logical(c_vec, 6)
                for pb in range(8):
                    g = plsc.load_gather(src, [r_vec, tb2[pb] + par])
                    plsc.store_scatter(dst, [pvs[pb], c_vec], g)

        for b in range(3):
            src_copy(32 * b + w, b).start()

        def group(q0, carry):
            for b in range(3):
                q = 3 * q0 + b
                dc = 32 * q + w

                @pl.when(dc < NDC)
                def _():
                    src_copy(dc, b).wait()

                    @pl.when(q >= 3)
                    def _():
                        dst_copy(dc - 96, b).wait()

                    transpose_block(srcs[b], dsts[b])
                    dst_copy(dc, b).start()

                    @pl.when(dc + 96 < NDC)
                    def _():
                        src_copy(dc + 96, b).start()

            return carry

        lax.fori_loop(0, 42, group, 0)

        qmax = lax.shift_right_logical(NDC - 1 - w, 5)
        for b in range(3):
            qb = qmax - (qmax - b) % 3

            @pl.when(qb >= 0)
            def _():
                dst_copy(32 * qb + w, b).wait()

        @pl.when(w == 4)
        def _():
            pltpu.sync_copy(
                table_hbm.at[:, pl.ds(128 * CT_FULL, 64)],
                srcp)

            pvs2 = (iota, iota + 16)

            @plsc.parallel_loop(0, 128, unroll=2)
            def _(c0):
                c_vec = (c0 + iota) & 127
                r_vec = c_vec & 63
                par = lax.shift_right_logical(c_vec, 6)
                for pb in range(2):
                    t_vec = lax.shift_left(pvs2[pb], 1) + par
                    g = plsc.load_gather(srcp, [r_vec, t_vec])
                    plsc.store_scatter(dstp, [pvs2[pb], c_vec], g)

            pltpu.sync_copy(dstp, scratch_hbm.at[pl.ds(64 * CT_FULL, 32)])

    return repack


@functools.lru_cache(maxsize=None)
def _make_gather(L, B):
    n_sb = (L // 8) * (B // 128)
    n_q = (n_sb + NW - 1) // NW

    @functools.partial(
        pl.kernel,
        mesh=_mesh(),
        out_type=jax.ShapeDtypeStruct((L, D, B), jnp.float32),
        compiler_params=_params,
        scratch_types=[
            pltpu.VMEM((8, 128), jnp.int32),
            pltpu.VMEM((8, 128), jnp.int32),
            pltpu.VMEM((8, 128), jnp.int32),
            pltpu.VMEM((128, 128), jnp.float32),
            pltpu.VMEM((128, 128), jnp.float32),
            pltpu.VMEM((128, 128), jnp.float32),
            pltpu.VMEM((64, 128), jnp.float32),
            pltpu.VMEM((64, 128), jnp.float32),
            pltpu.SemaphoreType.DMA,
            pltpu.SemaphoreType.DMA,
            pltpu.SemaphoreType.DMA,
            pltpu.SemaphoreType.DMA,
            pltpu.SemaphoreType.DMA,
        ],
    )
    def gather(scratch_hbm, ids_hbm, out_hbm, ids_v, idx2, par2,
               g0, g1, g2, o0, o1, gsem0, gsem1, gsem2, osem0, osem1):
        w = _wid()
        gbufs, obufs = (g0, g1, g2), (o0, o1)
        gsems, osems = (gsem0, gsem1, gsem2), (osem0, osem1)
        iota = _iota16()
        t_base = tuple(iota + 16 * m for m in range(8))

        def prep_all():
            @plsc.parallel_loop(0, 8)
            def _(i):
                for m in range(8):
                    v = ids_v[i, pl.ds(16 * m, 16)]
                    idx2[i, pl.ds(16 * m, 16)] = lax.shift_right_logical(v, 1)
                    par2[i, pl.ds(16 * m, 16)] = lax.shift_left(v & 1, 6)

        def gather_copy(i, bi):
            return pltpu.make_async_copy(
                scratch_hbm.at[idx2.at[i]], gbufs[bi], gsems[bi])

        def out_copy(l, bb, bi):
            return pltpu.make_async_copy(
                obufs[bi], out_hbm.at[l, :, pl.ds(bb * 128, 128)], osems[bi])

        def compute(i, bi, oi):
            pc = tuple(par2[i, pl.ds(16 * m, 16)] for m in range(8))

            @plsc.parallel_loop(0, 64, unroll=4)
            def _(d0):
                d_vec = (d0 + iota) & 63
                for m in range(8):
                    g = plsc.load_gather(
                        gbufs[bi], [t_base[m], pc[m] + d_vec])
                    plsc.store_scatter(obufs[oi], [d_vec, t_base[m]], g)

        def sb_body(q, carry):
            sb = 32 * q + w

            @pl.when(sb < n_sb)
            def _():
                l8 = lax.shift_right_logical(sb, 3)
                bb = sb & 7
                psb = sb - 32
                pl8 = lax.shift_right_logical(psb, 3)
                pbb = psb & 7
                pltpu.sync_copy(
                    ids_hbm.at[pl.ds(l8 * 8, 8), pl.ds(bb * 128, 128)], ids_v)
                prep_all()
                for i in range(3):
                    gather_copy(i, i).start()
                for i in range(8):
                    bi = i % 3
                    oi = i % 2
                    gather_copy(i, bi).wait()
                    if i >= 2:
                        out_copy(l8 * 8 + i - 2, bb, oi).wait()
                    else:

                        @pl.when(q > 0)
                        def _():
                            out_copy(pl8 * 8 + 6 + i, pbb, oi).wait()

                    compute(i, bi, oi)
                    out_copy(l8 * 8 + i, bb, oi).start()
                    if i + 3 < 8:
                        gather_copy(i + 3, bi).start()

            return carry

        lax.fori_loop(0, n_q, sb_body, 0)

        qlast = lax.shift_right_logical(n_sb - 1 - w, 5)
        lsb = 32 * qlast + w
        ll8 = lax.shift_right_logical(lsb, 3)
        lbb = lsb & 7
        for b in range(2):
            out_copy(ll8 * 8 + 6 + b, lbb, b).wait()

    return gather


def kernel(ids, emb_weight):
    batch, length = ids.shape
    table_t = emb_weight.T
    ids_t = ids.T
    scratch = _make_repack()(table_t)
    out_t = _make_gather(length, batch)(scratch, ids_t)
    return jnp.transpose(out_t, (2, 0, 1))


# speedup vs baseline: 6.6969x; 1.0009x over previous
import functools

import jax
import jax.numpy as jnp
from jax import lax
from jax.experimental import pallas as pl
from jax.experimental.pallas import tpu as pltpu
from jax.experimental.pallas import tpu_sc as plsc

V = 1_000_000
D = 64
NW = 32
CT_FULL = 7812
NPAIR = V // 2
SPAD = 128

_params = pltpu.CompilerParams(
    use_tc_tiling_on_sc=True, needs_layout_passes=False)


def _mesh():
    return plsc.VectorSubcoreMesh(core_axis_name="c", subcore_axis_name="s")


def _wid():
    info = plsc.get_sparse_core_info()
    return lax.axis_index("s") * info.num_cores + lax.axis_index("c")


def _iota16():
    return lax.iota(jnp.int32, 16)


@functools.lru_cache(maxsize=None)
def _make_repack():
    NDC = CT_FULL // 2

    @functools.partial(
        pl.kernel,
        mesh=_mesh(),
        out_type=jax.ShapeDtypeStruct((NPAIR, 128), jnp.float32),
        compiler_params=_params,
        scratch_types=[
            pltpu.VMEM((64, 256), jnp.float32),
            pltpu.VMEM((64, 256), jnp.float32),
            pltpu.VMEM((64, 256), jnp.float32),
            pltpu.VMEM((128, 128), jnp.float32),
            pltpu.VMEM((128, 128), jnp.float32),
            pltpu.VMEM((128, 128), jnp.float32),
            pltpu.VMEM((64, 64), jnp.float32),
            pltpu.VMEM((32, 128), jnp.float32),
            pltpu.SemaphoreType.DMA,
            pltpu.SemaphoreType.DMA,
            pltpu.SemaphoreType.DMA,
            pltpu.SemaphoreType.DMA,
            pltpu.SemaphoreType.DMA,
            pltpu.SemaphoreType.DMA,
        ],
    )
    def repack(table_hbm, scratch_hbm, src0, src1, src2a, dst0, dst1, dst2a,
               srcp, dstp, ssem0, ssem1, ssem2, dsem0, dsem1, dsem2):
        w = _wid()
        srcs, dsts = (src0, src1, src2a), (dst0, dst1, dst2a)
        ssems, dsems = (ssem0, ssem1, ssem2), (dsem0, dsem1, dsem2)
        iota = _iota16()

        def src_copy(dc, b):
            return pltpu.make_async_copy(
                table_hbm.at[:, pl.ds(dc * 256, 256)], srcs[b], ssems[b])

        def dst_copy(dc, b):
            return pltpu.make_async_copy(
                dsts[b], scratch_hbm.at[pl.ds(dc * 128, 128)], dsems[b])

        pvs = tuple(iota + 16 * pb for pb in range(8))
        tb2 = tuple(lax.shift_left(pv, 1) for pv in pvs)

        def transpose_block(src, dst):
            @plsc.parallel_loop(0, 128, unroll=4)
            def _(c0):
                c_vec = (c0 + iota) & 127
                r_vec = c_vec & 63
                par = lax.shift_right_logical(c_vec, 6)
                for pb in range(8):
                    g = plsc.load_gather(src, [r_vec, tb2[pb] + par])
                    plsc.store_scatter(dst, [pvs[pb], c_vec], g)

        for b in range(3):
            src_copy(32 * b + w, b).start()

        def group(q0, carry):
            for b in range(3):
                q = 3 * q0 + b
                dc = 32 * q + w

                @pl.when(dc < NDC)
                def _():
                    src_copy(dc, b).wait()

                    @pl.when(q >= 3)
                    def _():
                        dst_copy(dc - 96, b).wait()

                    transpose_block(srcs[b], dsts[b])
                    dst_copy(dc, b).start()

                    @pl.when(dc + 96 < NDC)
                    def _():
                        src_copy(dc + 96, b).start()

            return carry

        lax.fori_loop(0, 42, group, 0)

        qmax = lax.shift_right_logical(NDC - 1 - w, 5)
        for b in range(3):
            qb = qmax - (qmax - b) % 3

            @pl.when(qb >= 0)
            def _():
                dst_copy(32 * qb + w, b).wait()

        @pl.when(w == 4)
        def _():
            pltpu.sync_copy(
                table_hbm.at[:, pl.ds(128 * CT_FULL, 64)],
                srcp)

            pvs2 = (iota, iota + 16)

            @plsc.parallel_loop(0, 128, unroll=2)
            def _(c0):
                c_vec = (c0 + iota) & 127
                r_vec = c_vec & 63
                par = lax.shift_right_logical(c_vec, 6)
                for pb in range(2):
                    t_vec = lax.shift_left(pvs2[pb], 1) + par
                    g = plsc.load_gather(srcp, [r_vec, t_vec])
                    plsc.store_scatter(dstp, [pvs2[pb], c_vec], g)

            pltpu.sync_copy(dstp, scratch_hbm.at[pl.ds(64 * CT_FULL, 32)])

    return repack


@functools.lru_cache(maxsize=None)
def _make_gather(L, B):
    n_sb = (L // 8) * (B // 128)
    n_q = (n_sb + NW - 1) // NW

    @functools.partial(
        pl.kernel,
        mesh=_mesh(),
        out_type=jax.ShapeDtypeStruct((L, D, B), jnp.float32),
        compiler_params=_params,
        scratch_types=[
            pltpu.VMEM((8, 128), jnp.int32),
            pltpu.VMEM((8, 128), jnp.int32),
            pltpu.VMEM((8, 128), jnp.int32),
            pltpu.VMEM((128, 128), jnp.float32),
            pltpu.VMEM((128, 128), jnp.float32),
            pltpu.VMEM((128, 128), jnp.float32),
            pltpu.VMEM((128, 128), jnp.float32),
            pltpu.VMEM((64, 128), jnp.float32),
            pltpu.VMEM((64, 128), jnp.float32),
            pltpu.SemaphoreType.DMA,
            pltpu.SemaphoreType.DMA,
            pltpu.SemaphoreType.DMA,
            pltpu.SemaphoreType.DMA,
            pltpu.SemaphoreType.DMA,
            pltpu.SemaphoreType.DMA,
        ],
    )
    def gather(scratch_hbm, ids_hbm, out_hbm, ids_v, idx2, par2,
               g0, g1, g2, g3, o0, o1,
               gsem0, gsem1, gsem2, gsem3, osem0, osem1):
        w = _wid()
        gbufs, obufs = (g0, g1, g2, g3), (o0, o1)
        gsems, osems = (gsem0, gsem1, gsem2, gsem3), (osem0, osem1)
        iota = _iota16()
        t_base = tuple(iota + 16 * m for m in range(8))

        def prep_all():
            @plsc.parallel_loop(0, 8)
            def _(i):
                for m in range(8):
                    v = ids_v[i, pl.ds(16 * m, 16)]
                    idx2[i, pl.ds(16 * m, 16)] = lax.shift_right_logical(v, 1)
                    par2[i, pl.ds(16 * m, 16)] = lax.shift_left(v & 1, 6)

        def gather_copy(i, bi):
            return pltpu.make_async_copy(
                scratch_hbm.at[idx2.at[i]], gbufs[bi], gsems[bi])

        def out_copy(l, bb, bi):
            return pltpu.make_async_copy(
                obufs[bi], out_hbm.at[l, :, pl.ds(bb * 128, 128)], osems[bi])

        def compute(i, bi, oi):
            pc = tuple(par2[i, pl.ds(16 * m, 16)] for m in range(8))

            @plsc.parallel_loop(0, 64, unroll=4)
            def _(d0):
                d_vec = (d0 + iota) & 63
                for m in range(8):
                    g = plsc.load_gather(
                        gbufs[bi], [t_base[m], pc[m] + d_vec])
                    plsc.store_scatter(obufs[oi], [d_vec, t_base[m]], g)

        def sb_body(q, carry):
            sb = 32 * q + w

            @pl.when(sb < n_sb)
            def _():
                l8 = lax.shift_right_logical(sb, 3)
                bb = sb & 7
                psb = sb - 32
                pl8 = lax.shift_right_logical(psb, 3)
                pbb = psb & 7
                pltpu.sync_copy(
                    ids_hbm.at[pl.ds(l8 * 8, 8), pl.ds(bb * 128, 128)], ids_v)
                prep_all()
                for i in range(4):
                    gather_copy(i, i).start()
                for i in range(8):
                    bi = i % 4
                    oi = i % 2
                    gather_copy(i, bi).wait()
                    if i >= 2:
                        out_copy(l8 * 8 + i - 2, bb, oi).wait()
                    else:

                        @pl.when(q > 0)
                        def _():
                            out_copy(pl8 * 8 + 6 + i, pbb, oi).wait()

                    compute(i, bi, oi)
                    out_copy(l8 * 8 + i, bb, oi).start()
                    if i + 4 < 8:
                        gather_copy(i + 4, bi).start()

            return carry

        lax.fori_loop(0, n_q, sb_body, 0)

        qlast = lax.shift_right_logical(n_sb - 1 - w, 5)
        lsb = 32 * qlast + w
        ll8 = lax.shift_right_logical(lsb, 3)
        lbb = lsb & 7
        for b in range(2):
            out_copy(ll8 * 8 + 6 + b, lbb, b).wait()

    return gather


def kernel(ids, emb_weight):
    batch, length = ids.shape
    table_t = emb_weight.T
    ids_t = ids.T
    scratch = _make_repack()(table_t)
    out_t = _make_gather(length, batch)(scratch, ids_t)
    return jnp.transpose(out_t, (2, 0, 1))


# speedup vs baseline: 6.7097x; 1.0019x over previous
import functools

import jax
import jax.numpy as jnp
from jax import lax
from jax.experimental import pallas as pl
from jax.experimental.pallas import tpu as pltpu
from jax.experimental.pallas import tpu_sc as plsc

V = 1_000_000
D = 64
NW = 32
CT_FULL = 7812
NPAIR = V // 2

_params = pltpu.CompilerParams(
    use_tc_tiling_on_sc=True, needs_layout_passes=False)


def _mesh():
    return plsc.VectorSubcoreMesh(core_axis_name="c", subcore_axis_name="s")


def _wid():
    info = plsc.get_sparse_core_info()
    return lax.axis_index("s") * info.num_cores + lax.axis_index("c")


def _iota16():
    return lax.iota(jnp.int32, 16)


@functools.lru_cache(maxsize=None)
def _make_repack():
    NDC = CT_FULL // 2

    @functools.partial(
        pl.kernel,
        mesh=_mesh(),
        out_type=jax.ShapeDtypeStruct((NPAIR, 128), jnp.float32),
        compiler_params=_params,
        scratch_types=[
            pltpu.VMEM((64, 256), jnp.float32),
            pltpu.VMEM((64, 256), jnp.float32),
            pltpu.VMEM((64, 256), jnp.float32),
            pltpu.VMEM((128, 128), jnp.float32),
            pltpu.VMEM((128, 128), jnp.float32),
            pltpu.VMEM((128, 128), jnp.float32),
            pltpu.VMEM((64, 64), jnp.float32),
            pltpu.VMEM((32, 128), jnp.float32),
            pltpu.SemaphoreType.DMA,
            pltpu.SemaphoreType.DMA,
            pltpu.SemaphoreType.DMA,
            pltpu.SemaphoreType.DMA,
            pltpu.SemaphoreType.DMA,
            pltpu.SemaphoreType.DMA,
        ],
    )
    def repack(table_hbm, scratch_hbm, src0, src1, src2a, dst0, dst1, dst2a,
               srcp, dstp, ssem0, ssem1, ssem2, dsem0, dsem1, dsem2):
        w = _wid()
        srcs, dsts = (src0, src1, src2a), (dst0, dst1, dst2a)
        ssems, dsems = (ssem0, ssem1, ssem2), (dsem0, dsem1, dsem2)
        iota = _iota16()

        def src_copy(dc, b):
            return pltpu.make_async_copy(
                table_hbm.at[:, pl.ds(dc * 256, 256)], srcs[b], ssems[b])

        def dst_copy(dc, b):
            return pltpu.make_async_copy(
                dsts[b], scratch_hbm.at[pl.ds(dc * 128, 128)], dsems[b])

        pvs = tuple(iota + 16 * pb for pb in range(8))
        tb2 = tuple(lax.shift_left(pv, 1) for pv in pvs)

        def transpose_block(src, dst):
            @plsc.parallel_loop(0, 128, unroll=4)
            def _(c0):
                c_vec = (c0 + iota) & 127
                r_vec = c_vec & 63
                par = lax.shift_right_logical(c_vec, 6)
                for pb in range(8):
                    g = plsc.load_gather(src, [r_vec, tb2[pb] + par])
                    plsc.store_scatter(dst, [pvs[pb], c_vec], g)

        for b in range(3):
            src_copy(32 * b + w, b).start()

        def group(q0, carry):
            for b in range(3):
                q = 3 * q0 + b
                dc = 32 * q + w

                @pl.when(dc < NDC)
                def _():
                    src_copy(dc, b).wait()

                    @pl.when(q >= 3)
                    def _():
                        dst_copy(dc - 96, b).wait()

                    transpose_block(srcs[b], dsts[b])
                    dst_copy(dc, b).start()

                    @pl.when(dc + 96 < NDC)
                    def _():
                        src_copy(dc + 96, b).start()

            return carry

        lax.fori_loop(0, 42, group, 0)

        qmax = lax.shift_right_logical(NDC - 1 - w, 5)
        for b in range(3):
            qb = qmax - (qmax - b) % 3

            @pl.when(qb >= 0)
            def _():
                dst_copy(32 * qb + w, b).wait()

        @pl.when(w == 4)
        def _():
            pltpu.sync_copy(
                table_hbm.at[:, pl.ds(128 * CT_FULL, 64)],
                srcp)

            pvs2 = (iota, iota + 16)

            @plsc.parallel_loop(0, 128, unroll=2)
            def _(c0):
                c_vec = (c0 + iota) & 127
                r_vec = c_vec & 63
                par = lax.shift_right_logical(c_vec, 6)
                for pb in range(2):
                    t_vec = lax.shift_left(pvs2[pb], 1) + par
                    g = plsc.load_gather(srcp, [r_vec, t_vec])
                    plsc.store_scatter(dstp, [pvs2[pb], c_vec], g)

            pltpu.sync_copy(dstp, scratch_hbm.at[pl.ds(64 * CT_FULL, 32)])

    return repack


@functools.lru_cache(maxsize=None)
def _make_gather(L, B):
    n_sb = (L // 8) * (B // 128)
    n_q = (n_sb + NW - 1) // NW

    @functools.partial(
        pl.kernel,
        mesh=_mesh(),
        out_type=jax.ShapeDtypeStruct((L, D, B), jnp.float32),
        compiler_params=_params,
        scratch_types=[
            pltpu.VMEM((8, 128), jnp.int32),
            pltpu.VMEM((8, 128), jnp.int32),
            pltpu.VMEM((8, 128), jnp.int32),
            pltpu.VMEM((128, 128), jnp.float32),
            pltpu.VMEM((128, 128), jnp.float32),
            pltpu.VMEM((128, 128), jnp.float32),
            pltpu.VMEM((128, 128), jnp.float32),
            pltpu.VMEM((64, 128), jnp.float32),
            pltpu.VMEM((64, 128), jnp.float32),
            pltpu.SemaphoreType.DMA,
            pltpu.SemaphoreType.DMA,
            pltpu.SemaphoreType.DMA,
            pltpu.SemaphoreType.DMA,
            pltpu.SemaphoreType.DMA,
            pltpu.SemaphoreType.DMA,
        ],
    )
    def gather(scratch_hbm, ids_hbm, out_hbm, ids_v, idx2, par2,
               g0, g1, g2, g3, o0, o1,
               gsem0, gsem1, gsem2, gsem3, osem0, osem1):
        w = _wid()
        gbufs, obufs = (g0, g1, g2, g3), (o0, o1)
        gsems, osems = (gsem0, gsem1, gsem2, gsem3), (osem0, osem1)
        iota = _iota16()
        t_base = tuple(iota + 16 * m for m in range(8))

        def prep_all():
            @plsc.parallel_loop(0, 8)
            def _(i):
                for m in range(8):
                    v = ids_v[i, pl.ds(16 * m, 16)]
                    idx2[i, pl.ds(16 * m, 16)] = lax.shift_right_logical(v, 1)
                    par2[i, pl.ds(16 * m, 16)] = lax.shift_left(v & 1, 6)

        def gather_copy(i, bi):
            return pltpu.make_async_copy(
                scratch_hbm.at[idx2.at[i]], gbufs[bi], gsems[bi])

        def out_copy(l, bb, bi):
            return pltpu.make_async_copy(
                obufs[bi], out_hbm.at[l, :, pl.ds(bb * 128, 128)], osems[bi])

        def compute(i, bi, oi):
            pc = tuple(par2[i, pl.ds(16 * m, 16)] for m in range(8))

            @plsc.parallel_loop(0, 64, unroll=4)
            def _(d0):
                d_vec = (d0 + iota) & 63
                for m in range(8):
                    g = plsc.load_gather(
                        gbufs[bi], [t_base[m], pc[m] + d_vec])
                    plsc.store_scatter(obufs[oi], [d_vec, t_base[m]], g)

        def sb_body(q, carry):
            sb = 32 * q + w

            @pl.when(sb < n_sb)
            def _():
                l8 = lax.shift_right_logical(sb, 3)
                bb = sb & 7
                psb = sb - 32
                pl8 = lax.shift_right_logical(psb, 3)
                pbb = psb & 7
                pltpu.sync_copy(
                    ids_hbm.at[pl.ds(l8 * 8, 8), pl.ds(bb * 128, 128)], ids_v)
                prep_all()
                for i in range(4):
                    gather_copy(i, i).start()
                for i in range(8):
                    bi = i % 4
                    oi = i % 2
                    gather_copy(i, bi).wait()
                    if i >= 2:
                        out_copy(l8 * 8 + i - 2, bb, oi).wait()
                    else:

                        @pl.when(q > 0)
                        def _():
                            out_copy(pl8 * 8 + 6 + i, pbb, oi).wait()

                    compute(i, bi, oi)
                    out_copy(l8 * 8 + i, bb, oi).start()
                    if i + 4 < 8:
                        gather_copy(i + 4, bi).start()

            return carry

        lax.fori_loop(0, n_q, sb_body, 0)

        qlast = lax.shift_right_logical(n_sb - 1 - w, 5)
        lsb = 32 * qlast + w
        ll8 = lax.shift_right_logical(lsb, 3)
        lbb = lsb & 7
        for b in range(2):
            out_copy(ll8 * 8 + 6 + b, lbb, b).wait()

    return gather


def kernel(ids, emb_weight):
    batch, length = ids.shape
    table_t = emb_weight.T
    ids_t = ids.T
    scratch = _make_repack()(table_t)
    out_t = _make_gather(length, batch)(scratch, ids_t)
    return jnp.transpose(out_t, (2, 0, 1))
